# baseline (speedup 1.0000x reference)
import jax
import jax.numpy as jnp
from jax import lax
from jax.experimental import pallas as pl
from jax.experimental.pallas import tpu as pltpu


def _make_kernel(N, nhid, BR, NBUF):
    NBLK = N // BR

    def _gcn_kernel(x_ref, w_ref, b_ref, adj_ref, out_ref, buf_ref, support_ref, sems):
        support_ref[...] = jnp.dot(
            x_ref[...], w_ref[...], preferred_element_type=jnp.float32
        )

        def copy_in(blk, slot):
            return pltpu.make_async_copy(
                adj_ref.at[pl.ds(blk * BR, BR), :],
                buf_ref.at[slot],
                sems.at[slot],
            )

        for slot in range(NBUF):
            copy_in(slot, slot).start()

        def outer(j, carry):
            for slot in range(NBUF):
                blk = j * NBUF + slot
                copy_in(blk, slot).wait()
                block = buf_ref[slot]
                out = jnp.dot(
                    block, support_ref[...], preferred_element_type=jnp.float32
                )
                h = jnp.maximum(out + b_ref[...], 0.0)
                m = jnp.max(h, axis=1, keepdims=True)
                s = h - m
                lse = jnp.log(jnp.sum(jnp.exp(s), axis=1, keepdims=True))
                out_ref[pl.ds(blk * BR, BR), :] = s - lse

                @pl.when(blk + NBUF < NBLK)
                def _():
                    copy_in(blk + NBUF, slot).start()

            return carry

        lax.fori_loop(0, NBLK // NBUF, outer, 0, unroll=False)

    return _gcn_kernel


def kernel(x, adj, W, b):
    N, nfeat = x.shape
    nhid = W.shape[1]
    BR = 200
    NBUF = 5

    return pl.pallas_call(
        _make_kernel(N, nhid, BR, NBUF),
        in_specs=[
            pl.BlockSpec(memory_space=pltpu.MemorySpace.VMEM),
            pl.BlockSpec(memory_space=pltpu.MemorySpace.VMEM),
            pl.BlockSpec(memory_space=pltpu.MemorySpace.VMEM),
            pl.BlockSpec(memory_space=pltpu.MemorySpace.HBM),
        ],
        out_specs=pl.BlockSpec(memory_space=pltpu.MemorySpace.VMEM),
        out_shape=jax.ShapeDtypeStruct((N, nhid), jnp.float32),
        scratch_shapes=[
            pltpu.VMEM((NBUF, BR, N), jnp.float32),
            pltpu.VMEM((N, nhid), jnp.float32),
            pltpu.SemaphoreType.DMA((NBUF,)),
        ],
        compiler_params=pltpu.CompilerParams(
            vmem_limit_bytes=100 * 1024 * 1024,
        ),
    )(x, W, b.reshape(1, nhid), adj)
